# baseline (speedup 1.0000x reference)
import jax
import jax.numpy as jnp
from jax import lax
from jax.experimental import pallas as pl
from jax.experimental.pallas import tpu as pltpu

E = 16
D = 768
H = 3072
N = 32
NJ = 1
SPLIT = 8
HS = H // (NJ * SPLIT)


def _moe_kernel(*refs):
    (x_ref, wg_ref, bg_ref, b1_ref, b2_ref), w_refs, (out_ref,), \
        (w1s_ref, w2s_ref, i1s_ref, i2s_ref) = \
        refs[:5], refs[5:5 + 2 * SPLIT], refs[5 + 2 * SPLIT:6 + 2 * SPLIT], \
        refs[6 + 2 * SPLIT:]
    w1_refs = w_refs[:SPLIT]
    w2_refs = w_refs[SPLIT:]
    e = pl.program_id(0)
    j = pl.program_id(1)
    first = jnp.logical_and(e == 0, j == 0)

    @pl.when(first)
    def _gate():
        logits = lax.dot_general(
            x_ref[...], wg_ref[...], (((1,), (1,)), ((), ())),
            preferred_element_type=jnp.float32) + bg_ref[...]
        col = lax.broadcasted_iota(jnp.int32, (N, E), 1)
        m1 = jnp.max(logits, axis=-1, keepdims=True)
        i1 = jnp.min(jnp.where(logits == m1, col, E), axis=-1, keepdims=True)
        masked = jnp.where(col == i1, -jnp.inf, logits)
        m2 = jnp.max(masked, axis=-1, keepdims=True)
        i2 = jnp.min(jnp.where(masked == m2, col, E), axis=-1, keepdims=True)
        w1 = 1.0 / (1.0 + jnp.exp(m2 - m1))
        w1s_ref[...] = w1
        w2s_ref[...] = 1.0 - w1
        i1s_ref[...] = i1
        i2s_ref[...] = i2

    scol = (w1s_ref[...] * (i1s_ref[...] == e).astype(jnp.float32)
            + w2s_ref[...] * (i2s_ref[...] == e).astype(jnp.float32))

    xb = x_ref[...]
    acc = jnp.where(j == 0, scol * b2_ref[0], jnp.zeros((N, D), jnp.float32))
    for k in range(SPLIT):
        h = lax.dot_general(
            xb, w1_refs[k][0], (((1,), (1,)), ((), ())),
            preferred_element_type=jnp.float32) + b1_ref[0, k:k + 1, :]
        h = jnp.maximum(h, 0.0)
        hs = h * scol
        acc = acc + lax.dot_general(
            hs, w2_refs[k][0], (((1,), (1,)), ((), ())),
            preferred_element_type=jnp.float32)

    @pl.when(first)
    def _init():
        out_ref[...] = acc

    @pl.when(jnp.logical_not(first))
    def _acc():
        out_ref[...] = out_ref[...] + acc


@jax.jit
def _moe(x, Wg, bg2, W1, b1, W2, b2):
    w1_specs = [
        pl.BlockSpec((1, HS, D), lambda e, j, k=k: (e, j * SPLIT + k, 0))
        for k in range(SPLIT)
    ]
    w2_specs = [
        pl.BlockSpec((1, D, HS), lambda e, j, k=k: (e, 0, j * SPLIT + k))
        for k in range(SPLIT)
    ]
    return pl.pallas_call(
        _moe_kernel,
        grid=(E, NJ),
        in_specs=[
            pl.BlockSpec((N, D), lambda e, j: (0, 0)),
            pl.BlockSpec((E, D), lambda e, j: (0, 0)),
            pl.BlockSpec((1, E), lambda e, j: (0, 0)),
            pl.BlockSpec((1, SPLIT, HS), lambda e, j: (e * NJ + j, 0, 0)),
            pl.BlockSpec((1, 1, D), lambda e, j: (e, 0, 0)),
            *w1_specs,
            *w2_specs,
        ],
        out_specs=pl.BlockSpec((N, D), lambda e, j: (0, 0)),
        out_shape=jax.ShapeDtypeStruct((N, D), jnp.float32),
        scratch_shapes=[
            pltpu.VMEM((N, 1), jnp.float32),
            pltpu.VMEM((N, 1), jnp.float32),
            pltpu.VMEM((N, 1), jnp.int32),
            pltpu.VMEM((N, 1), jnp.int32),
        ],
        compiler_params=pltpu.CompilerParams(
            dimension_semantics=("arbitrary", "arbitrary"),
        ),
    )(x, Wg, bg2, b1, b2, *([W1] * SPLIT), *([W2] * SPLIT))


def kernel(x, Wg, bg, W1, b1, W2, b2):
    return _moe(x, Wg, bg.reshape(1, E),
                W1, b1.reshape(E * NJ, SPLIT, HS),
                W2, b2.reshape(E, 1, D))


# speedup vs baseline: 1.0012x; 1.0012x over previous
import jax
import jax.numpy as jnp
from jax import lax
from jax.experimental import pallas as pl
from jax.experimental.pallas import tpu as pltpu

E = 16
D = 768
H = 3072
N = 32
NJ = 1
SPLIT = 8
HS = H // (NJ * SPLIT)


def _moe_kernel(*refs):
    (x_ref, wg_ref, bg_ref, b1_ref, b2_ref), w_refs, (out_ref,), \
        (w1s_ref, w2s_ref, i1s_ref, i2s_ref) = \
        refs[:5], refs[5:5 + 2 * SPLIT], refs[5 + 2 * SPLIT:6 + 2 * SPLIT], \
        refs[6 + 2 * SPLIT:]
    w1_refs = w_refs[:SPLIT]
    w2_refs = w_refs[SPLIT:]
    e = pl.program_id(0)
    j = pl.program_id(1)
    first = jnp.logical_and(e == 0, j == 0)

    @pl.when(first)
    def _gate():
        logits = lax.dot_general(
            x_ref[...], wg_ref[...], (((1,), (1,)), ((), ())),
            preferred_element_type=jnp.float32) + bg_ref[...]
        col = lax.broadcasted_iota(jnp.int32, (N, E), 1)
        m1 = jnp.max(logits, axis=-1, keepdims=True)
        i1 = jnp.min(jnp.where(logits == m1, col, E), axis=-1, keepdims=True)
        masked = jnp.where(col == i1, -jnp.inf, logits)
        m2 = jnp.max(masked, axis=-1, keepdims=True)
        i2 = jnp.min(jnp.where(masked == m2, col, E), axis=-1, keepdims=True)
        w1 = 1.0 / (1.0 + jnp.exp(m2 - m1))
        w1s_ref[...] = w1
        w2s_ref[...] = 1.0 - w1
        i1s_ref[...] = i1
        i2s_ref[...] = i2

    scol = (w1s_ref[...] * (i1s_ref[...] == e).astype(jnp.float32)
            + w2s_ref[...] * (i2s_ref[...] == e).astype(jnp.float32))

    xb = x_ref[...].astype(jnp.bfloat16)
    acc = jnp.where(j == 0, scol * b2_ref[0], jnp.zeros((N, D), jnp.float32))
    for k in range(SPLIT):
        h = lax.dot_general(
            xb, w1_refs[k][0].astype(jnp.bfloat16), (((1,), (1,)), ((), ())),
            preferred_element_type=jnp.float32) + b1_ref[0, k:k + 1, :]
        h = jnp.maximum(h, 0.0)
        hs = (h * scol).astype(jnp.bfloat16)
        acc = acc + lax.dot_general(
            hs, w2_refs[k][0].astype(jnp.bfloat16), (((1,), (1,)), ((), ())),
            preferred_element_type=jnp.float32)

    @pl.when(first)
    def _init():
        out_ref[...] = acc

    @pl.when(jnp.logical_not(first))
    def _acc():
        out_ref[...] = out_ref[...] + acc


@jax.jit
def _moe(x, Wg, bg2, W1, b1, W2, b2):
    w1_specs = [
        pl.BlockSpec((1, HS, D), lambda e, j, k=k: (e, j * SPLIT + k, 0))
        for k in range(SPLIT)
    ]
    w2_specs = [
        pl.BlockSpec((1, D, HS), lambda e, j, k=k: (e, 0, j * SPLIT + k))
        for k in range(SPLIT)
    ]
    return pl.pallas_call(
        _moe_kernel,
        grid=(E, NJ),
        in_specs=[
            pl.BlockSpec((N, D), lambda e, j: (0, 0)),
            pl.BlockSpec((E, D), lambda e, j: (0, 0)),
            pl.BlockSpec((1, E), lambda e, j: (0, 0)),
            pl.BlockSpec((1, SPLIT, HS), lambda e, j: (e * NJ + j, 0, 0)),
            pl.BlockSpec((1, 1, D), lambda e, j: (e, 0, 0)),
            *w1_specs,
            *w2_specs,
        ],
        out_specs=pl.BlockSpec((N, D), lambda e, j: (0, 0)),
        out_shape=jax.ShapeDtypeStruct((N, D), jnp.float32),
        scratch_shapes=[
            pltpu.VMEM((N, 1), jnp.float32),
            pltpu.VMEM((N, 1), jnp.float32),
            pltpu.VMEM((N, 1), jnp.int32),
            pltpu.VMEM((N, 1), jnp.int32),
        ],
        compiler_params=pltpu.CompilerParams(
            dimension_semantics=("arbitrary", "arbitrary"),
        ),
    )(x, Wg, bg2, b1, b2, *([W1] * SPLIT), *([W2] * SPLIT))


def kernel(x, Wg, bg, W1, b1, W2, b2):
    return _moe(x, Wg, bg.reshape(1, E),
                W1, b1.reshape(E * NJ, SPLIT, HS),
                W2, b2.reshape(E, 1, D))
